# baseline (speedup 1.0000x reference)
import functools

import jax
import jax.numpy as jnp
from jax import lax
from jax.experimental import pallas as pl
from jax.experimental.pallas import tpu as pltpu
from jax.experimental.pallas import tpu_sc as plsc

_KEY_HI = 0
_KEY_LO = 42
_INV_KPROB = 2.0

_TC_BLOCK = 262144
_TC_CHUNK = 8192

_NW = 32
_SC_CHUNK = 4096
_SC_TOTAL = 0

_ROT_A = (13, 15, 26, 6)
_ROT_B = (17, 29, 16, 24)


def _threefry_keep(idx):
    ks0 = jnp.uint32(_KEY_HI)
    ks1 = jnp.uint32(_KEY_LO)
    ks2 = jnp.uint32(_KEY_HI ^ _KEY_LO ^ 0x1BD11BDA)

    x0 = jnp.zeros(idx.shape, jnp.uint32) + ks0
    x1 = idx + ks1

    def rotl(x, d):
        return (x << jnp.uint32(d)) | (x >> jnp.uint32(32 - d))

    injections = ((ks1, ks2), (ks2, ks0), (ks0, ks1), (ks1, ks2), (ks2, ks0))
    for i, (a, b) in enumerate(injections):
        for r in _ROT_A if i % 2 == 0 else _ROT_B:
            x0 = x0 + x1
            x1 = rotl(x1, r)
            x1 = x1 ^ x0
        x0 = x0 + a
        x1 = x1 + b + jnp.uint32(i + 1)

    bits = x0 ^ x1
    return (bits >> jnp.uint32(31)) == jnp.uint32(1)


def _tc_block_body(values_ref, indices_ref, out_ref, ind_out_ref):
    j = pl.program_id(0) + _SC_TOTAL // _TC_BLOCK
    rows = _TC_CHUNK // 8
    ind_out_ref[...] = indices_ref[...]
    for c in range(_TC_BLOCK // _TC_CHUNK):
        base = (j * _TC_BLOCK + c * _TC_CHUNK).astype(jnp.uint32)
        idx = (
            base
            + jax.lax.broadcasted_iota(jnp.uint32, (8, rows), 0) * rows
            + jax.lax.broadcasted_iota(jnp.uint32, (8, rows), 1)
        )
        keep = _threefry_keep(idx)
        v = values_ref[c * _TC_CHUNK : (c + 1) * _TC_CHUNK].reshape(8, rows)
        out = jnp.where(keep, v * _INV_KPROB, 0.0)
        out_ref[c * _TC_CHUNK : (c + 1) * _TC_CHUNK] = out.reshape(_TC_CHUNK)


def _tc_dropout(indices, values):
    nnz = values.shape[0]
    tail = nnz - _SC_TOTAL
    skip = _SC_TOTAL // _TC_BLOCK
    grid = pl.cdiv(tail, _TC_BLOCK)
    ind_block = -(-nnz // grid) & ~1023
    while ind_block * grid < nnz:
        ind_block += 1024
    out, ind_out = pl.pallas_call(
        _tc_block_body,
        grid=(grid,),
        in_specs=[
            pl.BlockSpec((_TC_BLOCK,), lambda j: (j + skip,)),
            pl.BlockSpec((2, ind_block), lambda j: (0, j)),
        ],
        out_specs=[
            pl.BlockSpec((_TC_BLOCK,), lambda j: (j,)),
            pl.BlockSpec((2, ind_block), lambda j: (0, j)),
        ],
        out_shape=[
            jax.ShapeDtypeStruct((tail,), values.dtype),
            jax.ShapeDtypeStruct((2, nnz), indices.dtype),
        ],
    )(values, indices)
    return out, ind_out


def _sc_dropout(values):
    per_w = _SC_TOTAL // _NW
    n_chunks = per_w // _SC_CHUNK
    mesh = plsc.VectorSubcoreMesh(core_axis_name="c", subcore_axis_name="s")

    @functools.partial(
        pl.kernel,
        mesh=mesh,
        out_type=jax.ShapeDtypeStruct((_SC_TOTAL,), jnp.float32),
        scratch_types=[
            pltpu.VMEM((2, _SC_CHUNK), jnp.float32),
            pltpu.VMEM((2, _SC_CHUNK), jnp.float32),
            pltpu.SemaphoreType.DMA,
            pltpu.SemaphoreType.DMA,
            pltpu.SemaphoreType.DMA,
            pltpu.SemaphoreType.DMA,
        ],
    )
    def k(values_hbm, out_hbm, vin, vout, sin0, sin1, sout0, sout1):
        wid = lax.axis_index("s") * 2 + lax.axis_index("c")
        base = wid * per_w
        sins = (sin0, sin1)
        souts = (sout0, sout1)

        def get_in(c, buf):
            return pltpu.make_async_copy(
                values_hbm.at[pl.ds(base + c * _SC_CHUNK, _SC_CHUNK)],
                vin.at[buf],
                sins[buf],
            )

        def put_out(c, buf):
            return pltpu.make_async_copy(
                vout.at[buf],
                out_hbm.at[pl.ds(base + c * _SC_CHUNK, _SC_CHUNK)],
                souts[buf],
            )

        get_in(0, 0).start()
        get_in(1, 1).start()
        lane = lax.iota(jnp.uint32, 16)

        def chunk_body(c, buf):
            get_in(c, buf).wait()

            @pl.when(c >= 2)
            def _():
                put_out(c - 2, buf).wait()

            cbase = (base + c * _SC_CHUNK).astype(jnp.uint32)

            def vec_body(q, _):
                idx = cbase + (q * 16).astype(jnp.uint32) + lane
                keep = _threefry_keep(idx)
                v = vin[buf, pl.ds(q * 16, 16)]
                vout[buf, pl.ds(q * 16, 16)] = jnp.where(keep, v * _INV_KPROB, 0.0)
                return ()

            lax.fori_loop(0, _SC_CHUNK // 16, vec_body, (), unroll=2)
            put_out(c, buf).start()

            @pl.when(c + 2 < n_chunks)
            def _():
                get_in(c + 2, buf).start()

        def pair_body(p, _):
            c = p * 2
            chunk_body(c, 0)
            chunk_body(c + 1, 1)
            return ()

        lax.fori_loop(0, n_chunks // 2, pair_body, ())
        put_out(n_chunks - 2, 0).wait()
        put_out(n_chunks - 1, 1).wait()

    return k(values)


@jax.jit
def _sparse_dropout(indices, values):
    if _SC_TOTAL == 0:
        tail, ind_out = _tc_dropout(indices, values)
        return ind_out, tail
    head = _sc_dropout(values)
    tail, ind_out = _tc_dropout(indices, values)
    return ind_out, jnp.concatenate([head, tail])


def kernel(indices, values):
    ind_out, new_values = _sparse_dropout(indices, values)
    return ind_out, new_values


# speedup vs baseline: 1.0480x; 1.0480x over previous
import jax
import jax.numpy as jnp
from jax.experimental import pallas as pl

_KEY_HI = 0
_KEY_LO = 42
_INV_KPROB = 2.0

_BLOCK = 131072
_CHUNK = 8192

_ROT_A = (13, 15, 26, 6)
_ROT_B = (17, 29, 16, 24)


def _threefry_keep(idx):
    ks0 = jnp.uint32(_KEY_HI)
    ks1 = jnp.uint32(_KEY_LO)
    ks2 = jnp.uint32(_KEY_HI ^ _KEY_LO ^ 0x1BD11BDA)

    x0 = jnp.zeros(idx.shape, jnp.uint32) + ks0
    x1 = idx + ks1

    def rotl(x, d):
        return (x << jnp.uint32(d)) | (x >> jnp.uint32(32 - d))

    injections = ((ks1, ks2), (ks2, ks0), (ks0, ks1), (ks1, ks2), (ks2, ks0))
    for i, (a, b) in enumerate(injections):
        for r in _ROT_A if i % 2 == 0 else _ROT_B:
            x0 = x0 + x1
            x1 = rotl(x1, r)
            x1 = x1 ^ x0
        x0 = x0 + a
        x1 = x1 + b + jnp.uint32(i + 1)

    bits = x0 ^ x1
    return (bits >> jnp.uint32(31)) == jnp.uint32(1)


def _block_body(values_ref, indices_ref, out_ref, ind_out_ref):
    j = pl.program_id(0)
    rows = _CHUNK // 8
    ind_out_ref[...] = indices_ref[...]
    for c in range(_BLOCK // _CHUNK):
        base = (j * _BLOCK + c * _CHUNK).astype(jnp.uint32)
        idx = (
            base
            + jax.lax.broadcasted_iota(jnp.uint32, (8, rows), 0) * rows
            + jax.lax.broadcasted_iota(jnp.uint32, (8, rows), 1)
        )
        keep = _threefry_keep(idx)
        v = values_ref[c * _CHUNK : (c + 1) * _CHUNK].reshape(8, rows)
        out = jnp.where(keep, v * _INV_KPROB, 0.0)
        out_ref[c * _CHUNK : (c + 1) * _CHUNK] = out.reshape(_CHUNK)


@jax.jit
def _sparse_dropout(indices, values):
    nnz = values.shape[0]
    grid = pl.cdiv(nnz, _BLOCK)
    ind_block = (-(-nnz // grid) + 1023) // 1024 * 1024
    out, ind_out = pl.pallas_call(
        _block_body,
        grid=(grid,),
        in_specs=[
            pl.BlockSpec((_BLOCK,), lambda j: (j,)),
            pl.BlockSpec((2, ind_block), lambda j: (0, j)),
        ],
        out_specs=[
            pl.BlockSpec((_BLOCK,), lambda j: (j,)),
            pl.BlockSpec((2, ind_block), lambda j: (0, j)),
        ],
        out_shape=[
            jax.ShapeDtypeStruct((nnz,), values.dtype),
            jax.ShapeDtypeStruct((2, nnz), indices.dtype),
        ],
    )(values, indices)
    return ind_out, out


def kernel(indices, values):
    ind_out, new_values = _sparse_dropout(indices, values)
    return ind_out, new_values
